# baseline (speedup 1.0000x reference)
import functools

import jax
import jax.numpy as jnp
from jax import lax
from jax.experimental import pallas as pl
from jax.experimental.pallas import tpu as pltpu
from jax.experimental.pallas import tpu_sc as plsc

VOCAB = 30522
HID = 1024
MAXLEN = 2048
BATCH = 4
NLANES = 16
NCORES = 2
NSUBCORES = 16
NWORKERS = NCORES * NSUBCORES
NBLK = MAXLEN // NWORKERS
GPOS = 8
NGRP = NBLK // GPOS
GROWS = GPOS * BATCH
HSLICES = HID // NLANES
NPAIR = NGRP // 2

_mesh = plsc.VectorSubcoreMesh(core_axis_name="c", subcore_axis_name="s")


@functools.partial(
    pl.kernel,
    out_type=jax.ShapeDtypeStruct((BATCH, MAXLEN, HID), jnp.float32),
    mesh=_mesh,
    scratch_types=[
        pltpu.VMEM((BATCH, NBLK), jnp.int32),
        pltpu.VMEM((BATCH, NBLK + NLANES), jnp.int32),
        pltpu.VMEM((2, HID), jnp.float32),
        pltpu.VMEM((2, GPOS, HID), jnp.float32),
        pltpu.VMEM((2, GROWS, HID), jnp.float32),
        pltpu.SemaphoreType.DMA,
        pltpu.SemaphoreType.DMA,
        pltpu.SemaphoreType.DMA,
        pltpu.SemaphoreType.DMA,
        pltpu.SemaphoreType.DMA,
        pltpu.SemaphoreType.DMA,
        pltpu.SemaphoreType.DMA,
    ],
)
def _embed(tokens_hbm, segments_hbm, table_hbm, segtab_hbm, pos_hbm,
           out_hbm, idx_v, seg_v, segtab_v, pos_v, tok_v,
           sem_pre, sem_g0, sem_g1, sem_o0, sem_o1, sem_pA, sem_pB):
    wid = lax.axis_index("s") * NCORES + lax.axis_index("c")
    l0 = wid * NBLK
    sem_g = (sem_g0, sem_g1)
    sem_o = (sem_o0, sem_o1)
    sem_p = (sem_pA, sem_pB)

    pre = []
    for b in range(BATCH):
        pre.append(pltpu.async_copy(
            tokens_hbm.at[b, pl.ds(l0, NBLK)], idx_v.at[b], sem_pre))
        pre.append(pltpu.async_copy(
            segments_hbm.at[b, pl.ds(l0, NBLK)],
            seg_v.at[b, pl.ds(0, NBLK)], sem_pre))
    pre.append(pltpu.async_copy(segtab_hbm, segtab_v, sem_pre))
    for cp in pre:
        cp.wait()

    def start_gathers(g, buf):
        for b in range(BATCH):
            pltpu.async_copy(
                table_hbm.at[idx_v.at[b, pl.ds(g * GPOS, GPOS)]],
                tok_v.at[buf, pl.ds(b * GPOS, GPOS)], sem_g[buf])

    def start_pos(g, slot):
        pltpu.async_copy(
            pos_hbm.at[pl.ds(l0 + g * GPOS, GPOS)], pos_v.at[slot],
            sem_p[slot])

    def start_scats(g, buf):
        for b in range(BATCH):
            pltpu.async_copy(
                tok_v.at[buf, pl.ds(b * GPOS, GPOS)],
                out_hbm.at[b, pl.ds(l0 + g * GPOS, GPOS)], sem_o[buf])

    def wait_gathers(buf):
        pltpu.make_async_copy(
            table_hbm.at[pl.ds(0, GROWS)], tok_v.at[buf], sem_g[buf]).wait()

    def wait_pos(slot):
        pltpu.make_async_copy(
            pos_hbm.at[pl.ds(0, GPOS)], pos_v.at[slot], sem_p[slot]).wait()

    def wait_scats(buf):
        pltpu.make_async_copy(
            tok_v.at[buf], out_hbm.at[0, pl.ds(0, GROWS)], sem_o[buf]).wait()

    def compute(g, buf, slot):
        def row_body(r, rcarry):
            sf = []
            for b in range(BATCH):
                s = seg_v[b, pl.ds(g * GPOS + r, NLANES)][0]
                sf.append(lax.broadcast(s.astype(jnp.float32), (NLANES,)))

            @plsc.parallel_loop(0, HSLICES, unroll=16)
            def _add(h):
                sl = pl.ds(h * NLANES, NLANES)
                sg0 = segtab_v[0, sl]
                sg1 = segtab_v[1, sl]
                p = pos_v[slot, r, sl]
                psg0 = p + sg0
                dps = sg1 - sg0
                for b in range(BATCH):
                    row = b * GPOS + r
                    t = tok_v[buf, row, sl]
                    tok_v[buf, row, sl] = t + psg0 + sf[b] * dps

            return rcarry

        lax.fori_loop(0, GPOS, row_body, 0)

    start_gathers(0, 0)
    start_pos(0, 0)

    def pair_body(k, carry):
        g0 = 2 * k
        g1 = 2 * k + 1

        @pl.when(k > 0)
        def _():
            wait_scats(1)
        start_gathers(g1, 1)
        start_pos(g1, 1)
        wait_gathers(0)
        wait_pos(0)
        compute(g0, 0, 0)
        start_scats(g0, 0)

        wait_scats(0)
        @pl.when(k + 1 < NPAIR)
        def _():
            start_gathers(g0 + 2, 0)
            start_pos(g0 + 2, 0)
        wait_gathers(1)
        wait_pos(1)
        compute(g1, 1, 1)
        start_scats(g1, 1)
        return carry

    lax.fori_loop(0, NPAIR, pair_body, 0)
    wait_scats(1)


def kernel(tokens, segments, token_table, segment_table, pos_weight):
    return _embed(tokens, segments, token_table, segment_table, pos_weight)


# speedup vs baseline: 1.2039x; 1.2039x over previous
import functools

import jax
import jax.numpy as jnp
from jax import lax
from jax.experimental import pallas as pl
from jax.experimental.pallas import tpu as pltpu
from jax.experimental.pallas import tpu_sc as plsc

VOCAB = 30522
HID = 1024
MAXLEN = 2048
BATCH = 4
NLANES = 16
NCORES = 2
NSUBCORES = 16
NWORKERS = NCORES * NSUBCORES
NBLK = MAXLEN // NWORKERS
GPOS = 8
NGRP = NBLK // GPOS
GROWS = GPOS * BATCH
HSLICES = HID // NLANES
NPAIR = NGRP // 2

_mesh = plsc.VectorSubcoreMesh(core_axis_name="c", subcore_axis_name="s")


@functools.partial(
    pl.kernel,
    out_type=jax.ShapeDtypeStruct((BATCH, MAXLEN, HID), jnp.float32),
    mesh=_mesh,
    scratch_types=[
        pltpu.VMEM((BATCH, NBLK), jnp.int32),
        pltpu.VMEM((BATCH, NBLK + NLANES), jnp.int32),
        pltpu.VMEM((2, HID), jnp.float32),
        pltpu.VMEM((2, GPOS, HID), jnp.float32),
        pltpu.VMEM((2, GROWS, HID), jnp.float32),
        pltpu.SemaphoreType.DMA,
        pltpu.SemaphoreType.DMA,
        pltpu.SemaphoreType.DMA,
        pltpu.SemaphoreType.DMA,
        pltpu.SemaphoreType.DMA,
        pltpu.SemaphoreType.DMA,
        pltpu.SemaphoreType.DMA,
    ],
)
def _embed(tokens_hbm, segments_hbm, table_hbm, segtab_hbm, pos_hbm,
           out_hbm, idx_v, seg_v, segtab_v, pos_v, tok_v,
           sem_pre, sem_g0, sem_g1, sem_o0, sem_o1, sem_pA, sem_pB):
    wid = lax.axis_index("s") * NCORES + lax.axis_index("c")
    l0 = wid * NBLK
    sem_g = (sem_g0, sem_g1)
    sem_o = (sem_o0, sem_o1)
    sem_p = (sem_pA, sem_pB)

    pre = []
    for b in range(BATCH):
        pre.append(pltpu.async_copy(
            tokens_hbm.at[b, pl.ds(l0, NBLK)], idx_v.at[b], sem_pre))
        pre.append(pltpu.async_copy(
            segments_hbm.at[b, pl.ds(l0, NBLK)],
            seg_v.at[b, pl.ds(0, NBLK)], sem_pre))
    pre.append(pltpu.async_copy(segtab_hbm, segtab_v, sem_pre))
    for cp in pre:
        cp.wait()

    def start_gathers(g, buf):
        for b in range(BATCH):
            pltpu.async_copy(
                table_hbm.at[idx_v.at[b, pl.ds(g * GPOS, GPOS)]],
                tok_v.at[buf, pl.ds(b * GPOS, GPOS)], sem_g[buf])

    def start_pos(g, slot):
        pltpu.async_copy(
            pos_hbm.at[pl.ds(l0 + g * GPOS, GPOS)], pos_v.at[slot],
            sem_p[slot])

    def start_scats(g, buf):
        for b in range(BATCH):
            pltpu.async_copy(
                tok_v.at[buf, pl.ds(b * GPOS, GPOS)],
                out_hbm.at[b, pl.ds(l0 + g * GPOS, GPOS)], sem_o[buf])

    def wait_gathers(buf):
        pltpu.make_async_copy(
            table_hbm.at[pl.ds(0, GROWS)], tok_v.at[buf], sem_g[buf]).wait()

    def wait_pos(slot):
        pltpu.make_async_copy(
            pos_hbm.at[pl.ds(0, GPOS)], pos_v.at[slot], sem_p[slot]).wait()

    def wait_scats(buf):
        pltpu.make_async_copy(
            tok_v.at[buf], out_hbm.at[0, pl.ds(0, GROWS)], sem_o[buf]).wait()

    def compute(g, buf, slot):
        @plsc.parallel_loop(0, GPOS, unroll=1)
        def row_body(r):
            sf = []
            for b in range(BATCH):
                s = seg_v[b, pl.ds(g * GPOS + r, NLANES)][0]
                sf.append(lax.broadcast(s.astype(jnp.float32), (NLANES,)))

            @plsc.parallel_loop(0, HSLICES, unroll=8)
            def _add(h):
                sl = pl.ds(h * NLANES, NLANES)
                sg0 = segtab_v[0, sl]
                sg1 = segtab_v[1, sl]
                p = pos_v[slot, r, sl]
                psg0 = p + sg0
                dps = sg1 - sg0
                for b in range(BATCH):
                    row = b * GPOS + r
                    t = tok_v[buf, row, sl]
                    tok_v[buf, row, sl] = t + psg0 + sf[b] * dps


    start_gathers(0, 0)
    start_pos(0, 0)

    def pair_body(k, carry):
        g0 = 2 * k
        g1 = 2 * k + 1

        @pl.when(k > 0)
        def _():
            wait_scats(1)
        start_gathers(g1, 1)
        start_pos(g1, 1)
        wait_gathers(0)
        wait_pos(0)
        compute(g0, 0, 0)
        start_scats(g0, 0)

        wait_scats(0)
        @pl.when(k + 1 < NPAIR)
        def _():
            start_gathers(g0 + 2, 0)
            start_pos(g0 + 2, 0)
        wait_gathers(1)
        wait_pos(1)
        compute(g1, 1, 1)
        start_scats(g1, 1)
        return carry

    lax.fori_loop(0, NPAIR, pair_body, 0)
    wait_scats(1)


def kernel(tokens, segments, token_table, segment_table, pos_weight):
    return _embed(tokens, segments, token_table, segment_table, pos_weight)
